# baseline (speedup 1.0000x reference)
import functools

import jax
import jax.numpy as jnp
from jax import lax
from jax.experimental import pallas as pl
from jax.experimental.pallas import tpu as pltpu
from jax.experimental.pallas import tpu_sc as plsc

NC = 2
NS = 16
NW = NC * NS
LANES = 16
CHUNK = 128
GROUP = 8

_MESH = plsc.VectorSubcoreMesh(core_axis_name="c", subcore_axis_name="s")
_SC_PARAMS = pltpu.CompilerParams(use_tc_tiling_on_sc=False)


def _make_deg_kernel(acc_rows, cpt):
    strip = acc_rows // NS

    @functools.partial(
        pl.kernel,
        out_type=jax.ShapeDtypeStruct((NC, acc_rows, LANES), jnp.float32),
        mesh=_MESH,
        compiler_params=_SC_PARAMS,
        scratch_types=[
            pltpu.VMEM((cpt, CHUNK), jnp.int32),
            pltpu.VMEM((CHUNK, LANES), jnp.float32),
            pltpu.VMEM_SHARED((acc_rows, LANES), jnp.float32),
        ],
    )
    def deg_kernel(dst_hbm, ones_hbm, zeros_hbm, out_hbm, didx_v, ones_v, acc_sh):
        cid = lax.axis_index("c")
        sid = lax.axis_index("s")
        wid = sid * NC + cid
        pltpu.sync_copy(dst_hbm.at[wid], didx_v)
        pltpu.sync_copy(ones_hbm, ones_v)
        pltpu.sync_copy(zeros_hbm, acc_sh.at[pl.ds(sid * strip, strip)])
        plsc.subcore_barrier()

        @pl.loop(0, cpt)
        def _(j):
            pltpu.sync_copy(ones_v, acc_sh.at[didx_v.at[j]], add=True)

        plsc.subcore_barrier()
        pltpu.sync_copy(
            acc_sh.at[pl.ds(sid * strip, strip)],
            out_hbm.at[cid, pl.ds(sid * strip, strip)],
        )

    return deg_kernel


def _make_agg_kernel(acc_rows, cpt, width):
    strip = acc_rows // NS

    @functools.partial(
        pl.kernel,
        out_type=jax.ShapeDtypeStruct((NC, acc_rows, width), jnp.float32),
        mesh=_MESH,
        compiler_params=_SC_PARAMS,
        scratch_types=[
            pltpu.VMEM((cpt, CHUNK), jnp.int32),
            pltpu.VMEM((cpt, CHUNK), jnp.int32),
            pltpu.VMEM((CHUNK, width), jnp.float32),
            pltpu.VMEM_SHARED((acc_rows, width), jnp.float32),
        ],
    )
    def agg_kernel(src_hbm, dst_hbm, hp_hbm, zeros_hbm, out_hbm,
                   sidx_v, didx_v, rows_v, acc_sh):
        cid = lax.axis_index("c")
        sid = lax.axis_index("s")
        wid = sid * NC + cid
        pltpu.sync_copy(src_hbm.at[wid], sidx_v)
        pltpu.sync_copy(dst_hbm.at[wid], didx_v)
        pltpu.sync_copy(zeros_hbm, acc_sh.at[pl.ds(sid * strip, strip)])
        plsc.subcore_barrier()

        @pl.loop(0, cpt)
        def _(j):
            pltpu.sync_copy(hp_hbm.at[sidx_v.at[j]], rows_v)

        plsc.subcore_barrier()
        pltpu.sync_copy(
            acc_sh.at[pl.ds(sid * strip, strip)],
            out_hbm.at[cid, pl.ds(sid * strip, strip)],
        )

    return agg_kernel


def _mm_body(x_ref, w_ref, o_ref):
    o_ref[...] = jnp.dot(x_ref[...], w_ref[...],
                         preferred_element_type=jnp.float32)


def _make_prep_body(n_nodes, hid):
    def prep_body(dp_ref, h1_ref, hp_ref, cc_ref):
        deg = 1.0 + dp_ref[0] + dp_ref[1]
        c16 = lax.rsqrt(deg)
        cc = jnp.broadcast_to(c16[:n_nodes, 0:1], (n_nodes, hid))
        cc_ref[...] = cc
        hp_ref[...] = h1_ref[...] * cc

    return prep_body


def _make_post_body(n_nodes):
    def post_body(agg_ref, cc_ref, h_ref, b_ref, w_ref, h2_ref, hp2_ref):
        s = agg_ref[0][:n_nodes] + agg_ref[1][:n_nodes]
        cc = cc_ref[...]
        h = h_ref[...]
        act = jnp.maximum(cc * s + cc * cc * h + b_ref[...], 0.0)
        nxt = jnp.dot(act, w_ref[...], preferred_element_type=jnp.float32)
        h2_ref[...] = nxt
        hp2_ref[...] = nxt * cc

    return post_body


def _make_post_final_body(n_nodes):
    def post_final_body(agg_ref, cc_ref, h_ref, b_ref, w_ref, b3_ref, o_ref):
        s = agg_ref[0][:n_nodes] + agg_ref[1][:n_nodes]
        cc = cc_ref[...]
        h = h_ref[...]
        act = jnp.maximum(cc * s + cc * cc * h + b_ref[...], 0.0)
        o_ref[...] = jnp.dot(act, w_ref[...],
                             preferred_element_type=jnp.float32) + b3_ref[...]

    return post_final_body


def kernel(x, edge_index, W1, b1, W2, b2, W3, b3):
    n_nodes, in_dim = x.shape
    hid = W1.shape[1]
    out_dim = W3.shape[1]
    n_edges = edge_index.shape[1]

    cpt = -(-n_edges // (NW * CHUNK))
    cpt = GROUP * (-(-cpt // GROUP))
    e_pad = NW * cpt * CHUNK
    acc_rows = NS * (-(-(n_nodes + 1) // (8 * NS)) * 8)

    ei = edge_index.astype(jnp.int32)
    pad = e_pad - n_edges
    src_p = jnp.concatenate(
        [ei[0], jnp.zeros((pad,), jnp.int32)]).reshape(NW, cpt, CHUNK)
    dst_p = jnp.concatenate(
        [ei[1], jnp.full((pad,), n_nodes, jnp.int32)]).reshape(NW, cpt, CHUNK)

    ones16 = jnp.ones((CHUNK, LANES), jnp.float32)
    zeros16 = jnp.zeros((acc_rows // NS, LANES), jnp.float32)
    zeros64 = jnp.zeros((acc_rows // NS, hid), jnp.float32)

    deg_k = _make_deg_kernel(acc_rows, cpt)
    agg_k = _make_agg_kernel(acc_rows, cpt, hid)

    f32 = jnp.float32

    h1 = pl.pallas_call(
        _mm_body,
        out_shape=jax.ShapeDtypeStruct((n_nodes, hid), f32),
    )(x, W1)

    dp = deg_k(dst_p, ones16, zeros16)

    hp1, cc = pl.pallas_call(
        _make_prep_body(n_nodes, hid),
        out_shape=(jax.ShapeDtypeStruct((n_nodes, hid), f32),
                   jax.ShapeDtypeStruct((n_nodes, hid), f32)),
    )(dp, h1)

    a1 = agg_k(src_p, dst_p, hp1, zeros64)

    h2, hp2 = pl.pallas_call(
        _make_post_body(n_nodes),
        out_shape=(jax.ShapeDtypeStruct((n_nodes, hid), f32),
                   jax.ShapeDtypeStruct((n_nodes, hid), f32)),
    )(a1, cc, h1, b1.reshape(1, hid), W2)

    a2 = agg_k(src_p, dst_p, hp2, zeros64)

    out = pl.pallas_call(
        _make_post_final_body(n_nodes),
        out_shape=jax.ShapeDtypeStruct((n_nodes, out_dim), f32),
    )(a2, cc, h2, b2.reshape(1, hid), W3, b3.reshape(1, out_dim))

    return out


# speedup vs baseline: 1.8124x; 1.8124x over previous
import functools

import jax
import jax.numpy as jnp
from jax import lax
from jax.experimental import pallas as pl
from jax.experimental.pallas import tpu as pltpu
from jax.experimental.pallas import tpu_sc as plsc

NC = 2
NS = 16
NW = NC * NS
LANES = 16
CHUNK = 128
GROUP = 8

_MESH = plsc.VectorSubcoreMesh(core_axis_name="c", subcore_axis_name="s")
_SC_PARAMS = pltpu.CompilerParams(use_tc_tiling_on_sc=False)


def _make_deg_kernel(acc_rows, cpt):
    strip = acc_rows // NS

    @functools.partial(
        pl.kernel,
        out_type=jax.ShapeDtypeStruct((NC, acc_rows, LANES), jnp.float32),
        mesh=_MESH,
        compiler_params=_SC_PARAMS,
        scratch_types=[
            pltpu.VMEM((cpt, CHUNK), jnp.int32),
            pltpu.VMEM((CHUNK, LANES), jnp.float32),
            pltpu.VMEM_SHARED((acc_rows, LANES), jnp.float32),
        ],
    )
    def deg_kernel(dst_hbm, ones_hbm, zeros_hbm, out_hbm, didx_v, ones_v, acc_sh):
        cid = lax.axis_index("c")
        sid = lax.axis_index("s")
        wid = sid * NC + cid
        pltpu.sync_copy(dst_hbm.at[wid], didx_v)
        pltpu.sync_copy(ones_hbm, ones_v)
        pltpu.sync_copy(zeros_hbm, acc_sh.at[pl.ds(sid * strip, strip)])
        plsc.subcore_barrier()

        @pl.loop(0, cpt)
        def _(j):
            pltpu.sync_copy(ones_v, acc_sh.at[didx_v.at[j]], add=True)

        plsc.subcore_barrier()
        pltpu.sync_copy(
            acc_sh.at[pl.ds(sid * strip, strip)],
            out_hbm.at[cid, pl.ds(sid * strip, strip)],
        )

    return deg_kernel


def _make_agg_kernel(acc_rows, cpt, width, n_nodes):
    strip = acc_rows // NS
    hp_strip = 8 * (-(-n_nodes // (NS * 8)))
    hp_tail_off = (NS - 1) * hp_strip
    hp_tail = n_nodes - hp_tail_off

    @functools.partial(
        pl.kernel,
        out_type=jax.ShapeDtypeStruct((NC, acc_rows, width), jnp.float32),
        mesh=_MESH,
        compiler_params=_SC_PARAMS,
        scratch_types=[
            pltpu.VMEM((cpt, CHUNK), jnp.int32),
            pltpu.VMEM((cpt, CHUNK), jnp.int32),
            pltpu.VMEM((CHUNK, width), jnp.float32),
            pltpu.VMEM_SHARED((n_nodes, width), jnp.float32),
            pltpu.VMEM_SHARED((acc_rows, width), jnp.float32),
        ],
    )
    def agg_kernel(src_hbm, dst_hbm, hp_hbm, zeros_hbm, out_hbm,
                   sidx_v, didx_v, rows_v, hp_sh, acc_sh):
        cid = lax.axis_index("c")
        sid = lax.axis_index("s")
        wid = sid * NC + cid
        pltpu.sync_copy(src_hbm.at[wid], sidx_v)
        pltpu.sync_copy(dst_hbm.at[wid], didx_v)
        pltpu.sync_copy(zeros_hbm, acc_sh.at[pl.ds(sid * strip, strip)])
        @pl.when(sid < NS - 1)
        def _():
            pltpu.sync_copy(hp_hbm.at[pl.ds(sid * hp_strip, hp_strip)],
                            hp_sh.at[pl.ds(sid * hp_strip, hp_strip)])

        @pl.when(sid == NS - 1)
        def _():
            pltpu.sync_copy(hp_hbm.at[pl.ds(hp_tail_off, hp_tail)],
                            hp_sh.at[pl.ds(hp_tail_off, hp_tail)])

        plsc.subcore_barrier()

        @pl.loop(0, cpt)
        def _(j):
            pltpu.sync_copy(hp_sh.at[sidx_v.at[j]], rows_v)
            pltpu.sync_copy(rows_v, acc_sh.at[didx_v.at[j]], add=True)

        plsc.subcore_barrier()
        pltpu.sync_copy(
            acc_sh.at[pl.ds(sid * strip, strip)],
            out_hbm.at[cid, pl.ds(sid * strip, strip)],
        )

    return agg_kernel


def _mm_body(x_ref, w_ref, o_ref):
    o_ref[...] = jnp.dot(x_ref[...], w_ref[...],
                         preferred_element_type=jnp.float32)


def _make_prep_body(n_nodes, hid):
    def prep_body(dp_ref, h1_ref, hp_ref, cc_ref):
        deg = 1.0 + dp_ref[0] + dp_ref[1]
        c16 = lax.rsqrt(deg)
        cc = jnp.broadcast_to(c16[:n_nodes, 0:1], (n_nodes, hid))
        cc_ref[...] = cc
        hp_ref[...] = h1_ref[...] * cc

    return prep_body


def _make_post_body(n_nodes):
    def post_body(agg_ref, cc_ref, h_ref, b_ref, w_ref, h2_ref, hp2_ref):
        s = agg_ref[0][:n_nodes] + agg_ref[1][:n_nodes]
        cc = cc_ref[...]
        h = h_ref[...]
        act = jnp.maximum(cc * s + cc * cc * h + b_ref[...], 0.0)
        nxt = jnp.dot(act, w_ref[...], preferred_element_type=jnp.float32)
        h2_ref[...] = nxt
        hp2_ref[...] = nxt * cc

    return post_body


def _make_post_final_body(n_nodes):
    def post_final_body(agg_ref, cc_ref, h_ref, b_ref, w_ref, b3_ref, o_ref):
        s = agg_ref[0][:n_nodes] + agg_ref[1][:n_nodes]
        cc = cc_ref[...]
        h = h_ref[...]
        act = jnp.maximum(cc * s + cc * cc * h + b_ref[...], 0.0)
        o_ref[...] = jnp.dot(act, w_ref[...],
                             preferred_element_type=jnp.float32) + b3_ref[...]

    return post_final_body


def kernel(x, edge_index, W1, b1, W2, b2, W3, b3):
    n_nodes, in_dim = x.shape
    hid = W1.shape[1]
    out_dim = W3.shape[1]
    n_edges = edge_index.shape[1]

    cpt = -(-n_edges // (NW * CHUNK))
    cpt = GROUP * (-(-cpt // GROUP))
    e_pad = NW * cpt * CHUNK
    acc_rows = NS * (-(-(n_nodes + 1) // (8 * NS)) * 8)

    ei = edge_index.astype(jnp.int32)
    pad = e_pad - n_edges
    src_p = jnp.concatenate(
        [ei[0], jnp.zeros((pad,), jnp.int32)]).reshape(NW, cpt, CHUNK)
    dst_p = jnp.concatenate(
        [ei[1], jnp.full((pad,), n_nodes, jnp.int32)]).reshape(NW, cpt, CHUNK)

    ones16 = jnp.ones((CHUNK, LANES), jnp.float32)
    zeros16 = jnp.zeros((acc_rows // NS, LANES), jnp.float32)
    zeros64 = jnp.zeros((acc_rows // NS, hid), jnp.float32)

    deg_k = _make_deg_kernel(acc_rows, cpt)
    agg_k = _make_agg_kernel(acc_rows, cpt, hid, n_nodes)

    f32 = jnp.float32

    h1 = pl.pallas_call(
        _mm_body,
        out_shape=jax.ShapeDtypeStruct((n_nodes, hid), f32),
    )(x, W1)

    dp = deg_k(dst_p, ones16, zeros16)

    hp1, cc = pl.pallas_call(
        _make_prep_body(n_nodes, hid),
        out_shape=(jax.ShapeDtypeStruct((n_nodes, hid), f32),
                   jax.ShapeDtypeStruct((n_nodes, hid), f32)),
    )(dp, h1)

    a1 = agg_k(src_p, dst_p, hp1, zeros64)

    h2, hp2 = pl.pallas_call(
        _make_post_body(n_nodes),
        out_shape=(jax.ShapeDtypeStruct((n_nodes, hid), f32),
                   jax.ShapeDtypeStruct((n_nodes, hid), f32)),
    )(a1, cc, h1, b1.reshape(1, hid), W2)

    a2 = agg_k(src_p, dst_p, hp2, zeros64)

    out = pl.pallas_call(
        _make_post_final_body(n_nodes),
        out_shape=jax.ShapeDtypeStruct((n_nodes, out_dim), f32),
    )(a2, cc, h2, b2.reshape(1, hid), W3, b3.reshape(1, out_dim))

    return out
